# baseline (speedup 1.0000x reference)
import functools

import jax
import jax.numpy as jnp
from jax import lax
from jax.experimental import pallas as pl
from jax.experimental.pallas import tpu as pltpu
from jax.experimental.pallas import tpu_sc as plsc

PHYSICS_W = 0.7
CONTEXT_W = 0.3
TOPK = 5
_NEG_INF = float("-inf")
_I32_MAX = jnp.iinfo(jnp.int32).max


def _l2_normalize(x):
    n = jnp.sqrt(jnp.sum(x * x, axis=-1, keepdims=True))
    return x / jnp.maximum(n, 1e-12)


def _score_topk_body(q_ref, qc_ref, keys_ref, ctx_ref, out_s_ref, out_i_ref,
                     ls_ref, is_ref, ps_ref, *, kt, nkt, ktotal, chunk):
    j = pl.program_id(0)
    nch = kt // chunk

    @pl.when(j == 0)
    def _init():
        ls_ref[...] = jnp.full(ls_ref.shape, _NEG_INF, jnp.float32)
        is_ref[...] = jnp.zeros(is_ref.shape, jnp.int32)
        ps_ref[...] = jnp.full(ps_ref.shape, _NEG_INF, jnp.float32)

    qn = q_ref[...]
    qcn = qc_ref[...]
    ktn = keys_ref[...]
    ctn = ctx_ref[...]

    dims = (((1,), (1,)), ((), ()))
    s = PHYSICS_W * lax.dot_general(qn, ktn, dims,
                                    preferred_element_type=jnp.float32)
    s = s + CONTEXT_W * lax.dot_general(qcn, ctn, dims,
                                        preferred_element_type=jnp.float32)

    qt = s.shape[0]
    L = [ls_ref[:, i * chunk:(i + 1) * chunk] for i in range(TOPK)]
    I = [is_ref[:, i * chunk:(i + 1) * chunk] for i in range(TOPK)]
    P = [ps_ref[:, i * chunk:(i + 1) * chunk] for i in range(TOPK)]
    lane = lax.broadcasted_iota(jnp.int32, (qt, chunk), 1)
    base = j * kt

    cbase = j * nch
    for pr in range(nch // 2):
        ch0 = 2 * pr
        idx0 = lane + (base + ch0 * chunk)
        c0 = s[:, ch0 * chunk:(ch0 + 1) * chunk]
        c0 = jnp.where(idx0 < ktotal, c0, _NEG_INF)
        c1 = s[:, (ch0 + 1) * chunk:(ch0 + 2) * chunk]
        c1 = jnp.where(idx0 + chunk < ktotal, c1, _NEG_INF)
        ge = c0 >= c1
        mx = jnp.maximum(c0, c1)
        mn = jnp.minimum(c0, c1)
        cidw = jnp.where(ge, cbase + ch0, cbase + ch0 + 1)
        gt = [mx > L[i] for i in range(TOPK)]
        newL = [jnp.where(gt[0], mx, L[0])]
        newI = [jnp.where(gt[0], cidw, I[0])]
        newP = [jnp.where(gt[0], mn, P[0])]
        for i in range(1, TOPK):
            newL.append(jnp.where(gt[i - 1], L[i - 1],
                                  jnp.where(gt[i], mx, L[i])))
            newI.append(jnp.where(gt[i - 1], I[i - 1],
                                  jnp.where(gt[i], cidw, I[i])))
            newP.append(jnp.where(gt[i - 1], P[i - 1],
                                  jnp.where(gt[i], mn, P[i])))
        L, I, P = newL, newI, newP

    for i in range(TOPK):
        ls_ref[:, i * chunk:(i + 1) * chunk] = L[i]
        is_ref[:, i * chunk:(i + 1) * chunk] = I[i]
        ps_ref[:, i * chunk:(i + 1) * chunk] = P[i]

    @pl.when(j == nkt - 1)
    def _emit():
        win = jnp.concatenate(I, axis=1)
        lanes = lax.broadcasted_iota(jnp.int32, (qt, TOPK * chunk), 1) % chunk
        cand = jnp.concatenate(L + P, axis=1)
        candi = jnp.concatenate(
            [win * chunk + lanes, (win ^ 1) * chunk + lanes], axis=1)
        for t in range(TOPK):
            m = jnp.max(cand, axis=1, keepdims=True)
            mi = jnp.min(jnp.where(cand == m, candi, _I32_MAX),
                         axis=1, keepdims=True)
            out_s_ref[:, t:t + 1] = m
            out_i_ref[:, t:t + 1] = mi
            cand = jnp.where((cand == m) & (candi == mi), _NEG_INF, cand)


def _score_topk(qn, qcn, kn, cn, *, kt=2560, chunk=128):
    q, d = qn.shape
    k, _ = kn.shape
    p = qcn.shape[1]
    nkt = -(-k // kt)

    body = functools.partial(_score_topk_body, kt=kt, nkt=nkt, ktotal=k,
                             chunk=chunk)
    out_s, out_i = pl.pallas_call(
        body,
        grid=(nkt,),
        in_specs=[
            pl.BlockSpec((q, d), lambda j: (0, 0)),
            pl.BlockSpec((q, p), lambda j: (0, 0)),
            pl.BlockSpec((kt, d), lambda j: (j, 0)),
            pl.BlockSpec((kt, p), lambda j: (j, 0)),
        ],
        out_specs=[
            pl.BlockSpec((q, TOPK), lambda j: (0, 0)),
            pl.BlockSpec((q, TOPK), lambda j: (0, 0)),
        ],
        out_shape=[
            jax.ShapeDtypeStruct((q, TOPK), jnp.float32),
            jax.ShapeDtypeStruct((q, TOPK), jnp.int32),
        ],
        scratch_shapes=[
            pltpu.VMEM((q, TOPK * chunk), jnp.float32),
            pltpu.VMEM((q, TOPK * chunk), jnp.int32),
            pltpu.VMEM((q, TOPK * chunk), jnp.float32),
        ],
    )(qn, qcn, kn, cn)
    return out_s, out_i


def _sc_gather(keys, ruls, sohs, idx_flat):
    b = idx_flat.shape[0]
    d = keys.shape[1]
    info = plsc.get_sparse_core_info()
    nw = info.num_cores * info.num_subcores
    bw = b // nw
    assert b % (8 * nw) == 0
    mesh = plsc.VectorSubcoreMesh(core_axis_name="c", subcore_axis_name="s")

    @functools.partial(
        pl.kernel,
        out_type=[
            jax.ShapeDtypeStruct((b, d), jnp.float32),
            jax.ShapeDtypeStruct((b,), jnp.float32),
            jax.ShapeDtypeStruct((b,), jnp.float32),
        ],
        mesh=mesh,
        scratch_types=[
            pltpu.VMEM((bw,), jnp.int32),
            pltpu.VMEM((bw, d), jnp.float32),
            pltpu.VMEM((bw,), jnp.float32),
            pltpu.VMEM((bw,), jnp.float32),
            pltpu.SemaphoreType.DMA,
        ],
    )
    def gather_kernel(keys_hbm, ruls_hbm, sohs_hbm, idx_hbm,
                      keys_out, ruls_out, sohs_out,
                      idx_v, rows_v, r_v, s_v, sem):
        wid = lax.axis_index("s") * info.num_cores + lax.axis_index("c")
        base = wid * bw
        pltpu.sync_copy(idx_hbm.at[pl.ds(base, bw)], idx_v)
        pltpu.async_copy(keys_hbm.at[idx_v], rows_v, sem).wait()
        pltpu.async_copy(ruls_hbm.at[idx_v], r_v, sem).wait()
        pltpu.async_copy(sohs_hbm.at[idx_v], s_v, sem).wait()
        pltpu.sync_copy(rows_v, keys_out.at[pl.ds(base, bw)])
        pltpu.sync_copy(r_v, ruls_out.at[pl.ds(base, bw)])
        pltpu.sync_copy(s_v, sohs_out.at[pl.ds(base, bw)])

    return gather_kernel(keys, ruls, sohs, idx_flat)


def kernel(query_latent, query_context, keys, contexts, ruls, sohs, k):
    q = query_latent.shape[0]
    d = keys.shape[1]
    bf = jnp.bfloat16
    qn = _l2_normalize(query_latent).astype(bf)
    qcn = _l2_normalize(query_context).astype(bf)
    kn = _l2_normalize(keys).astype(bf)
    cn = _l2_normalize(contexts).astype(bf)
    topk_scores, topk_idx = _score_topk(qn, qcn, kn, cn)
    rk_flat, rr_flat, rs_flat = _sc_gather(keys, ruls, sohs,
                                           topk_idx.reshape(-1))
    retrieved_keys = rk_flat.reshape(q, TOPK, d)
    retrieved_ruls = rr_flat.reshape(q, TOPK)
    retrieved_sohs = rs_flat.reshape(q, TOPK)
    return retrieved_keys, retrieved_ruls, retrieved_sohs, topk_scores


# speedup vs baseline: 1.0319x; 1.0319x over previous
import functools

import jax
import jax.numpy as jnp
from jax import lax
from jax.experimental import pallas as pl
from jax.experimental.pallas import tpu as pltpu
from jax.experimental.pallas import tpu_sc as plsc

PHYSICS_W = 0.7
CONTEXT_W = 0.3
TOPK = 5
_NEG_INF = float("-inf")
_I32_MAX = jnp.iinfo(jnp.int32).max


def _l2_normalize(x):
    n = jnp.sqrt(jnp.sum(x * x, axis=-1, keepdims=True))
    return x / jnp.maximum(n, 1e-12)


def _score_topk_body(q_ref, qc_ref, keys_ref, ctx_ref, out_s_ref, out_i_ref,
                     ls_ref, is_ref, ps_ref, *, kt, nkt, ktotal, chunk):
    j = pl.program_id(0)
    nch = kt // chunk

    @pl.when(j == 0)
    def _init():
        ls_ref[...] = jnp.full(ls_ref.shape, _NEG_INF, jnp.float32)
        is_ref[...] = jnp.zeros(is_ref.shape, jnp.int32)
        ps_ref[...] = jnp.full(ps_ref.shape, _NEG_INF, jnp.float32)

    qn = q_ref[...]
    qcn = qc_ref[...]
    ktn = keys_ref[...]
    ctn = ctx_ref[...]

    dims = (((1,), (1,)), ((), ()))
    s = PHYSICS_W * lax.dot_general(qn, ktn, dims,
                                    preferred_element_type=jnp.float32)
    s = s + CONTEXT_W * lax.dot_general(qcn, ctn, dims,
                                        preferred_element_type=jnp.float32)

    qt = s.shape[0]
    L = [ls_ref[:, i * chunk:(i + 1) * chunk] for i in range(TOPK)]
    I = [is_ref[:, i * chunk:(i + 1) * chunk] for i in range(TOPK)]
    P = [ps_ref[:, i * chunk:(i + 1) * chunk] for i in range(TOPK)]
    lane = lax.broadcasted_iota(jnp.int32, (qt, chunk), 1)
    base = j * kt

    cbase = j * nch
    t0 = ktotal - base
    for pr in range(nch // 2):
        ch0 = 2 * pr
        c0 = s[:, ch0 * chunk:(ch0 + 1) * chunk]
        c0 = jnp.where(lane < t0 - ch0 * chunk, c0, _NEG_INF)
        c1 = s[:, (ch0 + 1) * chunk:(ch0 + 2) * chunk]
        c1 = jnp.where(lane < t0 - (ch0 + 1) * chunk, c1, _NEG_INF)
        ge = c0 >= c1
        mx = jnp.maximum(c0, c1)
        mn = jnp.minimum(c0, c1)
        cidw = jnp.where(ge, cbase + ch0, cbase + ch0 + 1)
        gt = [mx > L[i] for i in range(TOPK)]
        newL = [jnp.where(gt[0], mx, L[0])]
        newI = [jnp.where(gt[0], cidw, I[0])]
        newP = [jnp.where(gt[0], mn, P[0])]
        for i in range(1, TOPK):
            newL.append(jnp.where(gt[i - 1], L[i - 1],
                                  jnp.where(gt[i], mx, L[i])))
            newI.append(jnp.where(gt[i - 1], I[i - 1],
                                  jnp.where(gt[i], cidw, I[i])))
            newP.append(jnp.where(gt[i - 1], P[i - 1],
                                  jnp.where(gt[i], mn, P[i])))
        L, I, P = newL, newI, newP

    for i in range(TOPK):
        ls_ref[:, i * chunk:(i + 1) * chunk] = L[i]
        is_ref[:, i * chunk:(i + 1) * chunk] = I[i]
        ps_ref[:, i * chunk:(i + 1) * chunk] = P[i]

    @pl.when(j == nkt - 1)
    def _emit():
        win = jnp.concatenate(I, axis=1)
        lanes = lax.broadcasted_iota(jnp.int32, (qt, TOPK * chunk), 1) % chunk
        cand = jnp.concatenate(L + P, axis=1)
        candi = jnp.concatenate(
            [win * chunk + lanes, (win ^ 1) * chunk + lanes], axis=1)
        for t in range(TOPK):
            m = jnp.max(cand, axis=1, keepdims=True)
            mi = jnp.min(jnp.where(cand == m, candi, _I32_MAX),
                         axis=1, keepdims=True)
            out_s_ref[:, t:t + 1] = m
            out_i_ref[:, t:t + 1] = mi
            cand = jnp.where((cand == m) & (candi == mi), _NEG_INF, cand)


def _score_topk(qn, qcn, kn, cn, *, kt=2048, chunk=128):
    q, d = qn.shape
    k, _ = kn.shape
    p = qcn.shape[1]
    nkt = -(-k // kt)

    body = functools.partial(_score_topk_body, kt=kt, nkt=nkt, ktotal=k,
                             chunk=chunk)
    out_s, out_i = pl.pallas_call(
        body,
        grid=(nkt,),
        in_specs=[
            pl.BlockSpec((q, d), lambda j: (0, 0)),
            pl.BlockSpec((q, p), lambda j: (0, 0)),
            pl.BlockSpec((kt, d), lambda j: (j, 0)),
            pl.BlockSpec((kt, p), lambda j: (j, 0)),
        ],
        out_specs=[
            pl.BlockSpec((q, TOPK), lambda j: (0, 0)),
            pl.BlockSpec((q, TOPK), lambda j: (0, 0)),
        ],
        out_shape=[
            jax.ShapeDtypeStruct((q, TOPK), jnp.float32),
            jax.ShapeDtypeStruct((q, TOPK), jnp.int32),
        ],
        scratch_shapes=[
            pltpu.VMEM((q, TOPK * chunk), jnp.float32),
            pltpu.VMEM((q, TOPK * chunk), jnp.int32),
            pltpu.VMEM((q, TOPK * chunk), jnp.float32),
        ],
    )(qn, qcn, kn, cn)
    return out_s, out_i


def _sc_gather(keys, ruls, sohs, idx_flat):
    b = idx_flat.shape[0]
    d = keys.shape[1]
    info = plsc.get_sparse_core_info()
    nw = info.num_cores * info.num_subcores
    bw = b // nw
    assert b % (8 * nw) == 0
    mesh = plsc.VectorSubcoreMesh(core_axis_name="c", subcore_axis_name="s")

    @functools.partial(
        pl.kernel,
        out_type=[
            jax.ShapeDtypeStruct((b, d), jnp.float32),
            jax.ShapeDtypeStruct((b,), jnp.float32),
            jax.ShapeDtypeStruct((b,), jnp.float32),
        ],
        mesh=mesh,
        scratch_types=[
            pltpu.VMEM((bw,), jnp.int32),
            pltpu.VMEM((bw, d), jnp.float32),
            pltpu.VMEM((bw,), jnp.float32),
            pltpu.VMEM((bw,), jnp.float32),
            pltpu.SemaphoreType.DMA,
        ],
    )
    def gather_kernel(keys_hbm, ruls_hbm, sohs_hbm, idx_hbm,
                      keys_out, ruls_out, sohs_out,
                      idx_v, rows_v, r_v, s_v, sem):
        wid = lax.axis_index("s") * info.num_cores + lax.axis_index("c")
        base = wid * bw
        pltpu.sync_copy(idx_hbm.at[pl.ds(base, bw)], idx_v)
        pltpu.async_copy(keys_hbm.at[idx_v], rows_v, sem).wait()
        pltpu.async_copy(ruls_hbm.at[idx_v], r_v, sem).wait()
        pltpu.async_copy(sohs_hbm.at[idx_v], s_v, sem).wait()
        pltpu.sync_copy(rows_v, keys_out.at[pl.ds(base, bw)])
        pltpu.sync_copy(r_v, ruls_out.at[pl.ds(base, bw)])
        pltpu.sync_copy(s_v, sohs_out.at[pl.ds(base, bw)])

    return gather_kernel(keys, ruls, sohs, idx_flat)


def kernel(query_latent, query_context, keys, contexts, ruls, sohs, k):
    q = query_latent.shape[0]
    d = keys.shape[1]
    bf = jnp.bfloat16
    qn = _l2_normalize(query_latent).astype(bf)
    qcn = _l2_normalize(query_context).astype(bf)
    kn = _l2_normalize(keys).astype(bf)
    cn = _l2_normalize(contexts).astype(bf)
    topk_scores, topk_idx = _score_topk(qn, qcn, kn, cn)
    rk_flat, rr_flat, rs_flat = _sc_gather(keys, ruls, sohs,
                                           topk_idx.reshape(-1))
    retrieved_keys = rk_flat.reshape(q, TOPK, d)
    retrieved_ruls = rr_flat.reshape(q, TOPK)
    retrieved_sohs = rs_flat.reshape(q, TOPK)
    return retrieved_keys, retrieved_ruls, retrieved_sohs, topk_scores
